# Initial kernel scaffold; baseline (speedup 1.0000x reference)
import jax
import jax.numpy as jnp
from jax.experimental import pallas as pl


def kernel(feats, ancs, ancs_valid, W1, b1, Wc, bc, Wr, br):
    raise NotImplementedError("write your pallas kernel here")


# speedup vs baseline: 13.4000x; 13.4000x over previous
import jax
import jax.numpy as jnp
from jax import lax
from jax.experimental import pallas as pl
from jax.experimental.pallas import tpu as pltpu

_NUM_ANCS = 9
_PRE = 6000
_POST = 300
_IOU_T = 0.7
_HW = 4096
_N = _HW * _NUM_ANCS
_ROWS = _N // 128
_TM = 512


def _mm_body(x_ref, w1_ref, b1_ref, wh_ref, bh_ref, y_ref):
    h = jnp.dot(x_ref[...], w1_ref[...], preferred_element_type=jnp.float32)
    h = jnp.maximum(h + b1_ref[...], 0.0)
    y = jnp.dot(h, wh_ref[...], preferred_element_type=jnp.float32) + bh_ref[...]
    y_ref[:, 0:128] = jax.nn.sigmoid(y[:, 0:128])
    y_ref[:, 128:640] = y[:, 128:640]


def _nms_body(s_ref, a0_ref, a1_ref, a2_ref, a3_ref, r0_ref, r1_ref, r2_ref,
              r3_ref, ox1_ref, oy1_ref, ox2_ref, oy2_ref,
              ms_ref, x1_ref, y1_ref, x2_ref, y2_ref, ar_ref):
    a2 = a2_ref[...]
    a3 = a3_ref[...]
    cx = a0_ref[...] + r0_ref[...] * a2
    cy = a1_ref[...] + r1_ref[...] * a3
    w = a2 * jnp.exp(r2_ref[...])
    h = a3 * jnp.exp(r3_ref[...])
    x1 = jnp.clip(cx - w * 0.5, 0.0, 1.0)
    y1 = jnp.clip(cy - h * 0.5, 0.0, 1.0)
    x2 = jnp.clip(cx + w * 0.5, 0.0, 1.0)
    y2 = jnp.clip(cy + h * 0.5, 0.0, 1.0)
    x1_ref[...] = x1
    y1_ref[...] = y1
    x2_ref[...] = x2
    y2_ref[...] = y2
    ar_ref[...] = (x2 - x1) * (y2 - y1)

    s = s_ref[...]
    bits = lax.bitcast_convert_type(s, jnp.int32)
    idx = (lax.broadcasted_iota(jnp.int32, (_ROWS, 128), 0) * 128
           + lax.broadcasted_iota(jnp.int32, (_ROWS, 128), 1))
    tau = jnp.int32(0)
    for b in range(30, -1, -1):
        cand = tau | jnp.int32(1 << b)
        cnt = jnp.sum((bits >= cand).astype(jnp.int32))
        tau = jnp.where(cnt >= _PRE, cand, tau)
    cnt_gt = jnp.sum((bits > tau).astype(jnp.int32))
    tie = bits == tau
    need = _PRE - cnt_gt
    theta = jnp.int32(0)
    for b in range(15, -1, -1):
        cand = theta | jnp.int32(1 << b)
        cnt = jnp.sum((tie & (idx >= cand)).astype(jnp.int32))
        theta = jnp.where(cnt >= need, cand, theta)
    active = (bits > tau) | (tie & (idx >= theta))
    ms_ref[...] = jnp.where(active, s, -1.0)

    def zero_body(k, c):
        ox1_ref[k] = 0.0
        oy1_ref[k] = 0.0
        ox2_ref[k] = 0.0
        oy2_ref[k] = 0.0
        return c

    lax.fori_loop(0, _POST, zero_body, 0)

    def nms_step(k, c):
        ms = ms_ref[...]
        m = jnp.max(ms)

        @pl.when(m >= 0.0)
        def _():
            bi = jnp.max(jnp.where(ms == m, idx, -1))
            eq = idx == bi
            bx1 = jnp.sum(jnp.where(eq, x1_ref[...], 0.0))
            by1 = jnp.sum(jnp.where(eq, y1_ref[...], 0.0))
            bx2 = jnp.sum(jnp.where(eq, x2_ref[...], 0.0))
            by2 = jnp.sum(jnp.where(eq, y2_ref[...], 0.0))
            ix1 = jnp.maximum(x1_ref[...], bx1)
            iy1 = jnp.maximum(y1_ref[...], by1)
            ix2 = jnp.minimum(x2_ref[...], bx2)
            iy2 = jnp.minimum(y2_ref[...], by2)
            inter = (jnp.maximum(ix2 - ix1, 0.0) * jnp.maximum(iy2 - iy1, 0.0))
            barea = (bx2 - bx1) * (by2 - by1)
            union = jnp.maximum(barea + ar_ref[...] - inter, 1e-8)
            supp = inter > _IOU_T * union
            ms_ref[...] = jnp.where(supp, -1.0, ms)
            ox1_ref[k] = bx1
            oy1_ref[k] = by1
            ox2_ref[k] = bx2
            oy2_ref[k] = by2

        return c

    lax.fori_loop(0, _POST, nms_step, 0)


def kernel(feats, ancs, ancs_valid, W1, b1, Wc, bc, Wr, br):
    del ancs_valid
    x = feats.reshape(_HW, 1024)

    wh = jnp.zeros((512, 640), jnp.float32)
    bh = jnp.zeros((640,), jnp.float32)
    wh = wh.at[:, 0:_NUM_ANCS].set(Wc)
    bh = bh.at[0:_NUM_ANCS].set(bc)
    for c in range(4):
        wh = wh.at[:, 128 * (c + 1):128 * (c + 1) + _NUM_ANCS].set(Wr[:, c::4])
        bh = bh.at[128 * (c + 1):128 * (c + 1) + _NUM_ANCS].set(br[c::4])

    y = pl.pallas_call(
        _mm_body,
        grid=(_HW // _TM,),
        in_specs=[
            pl.BlockSpec((_TM, 1024), lambda i: (i, 0)),
            pl.BlockSpec((1024, 512), lambda i: (0, 0)),
            pl.BlockSpec((1, 512), lambda i: (0, 0)),
            pl.BlockSpec((512, 640), lambda i: (0, 0)),
            pl.BlockSpec((1, 640), lambda i: (0, 0)),
        ],
        out_specs=pl.BlockSpec((_TM, 640), lambda i: (i, 0)),
        out_shape=jax.ShapeDtypeStruct((_HW, 640), jnp.float32),
    )(x, W1, b1.reshape(1, 512), wh, bh.reshape(1, 640))

    cls_pred = y[:, 0:_NUM_ANCS]
    reg_planes = [y[:, 128 * (c + 1):128 * (c + 1) + _NUM_ANCS] for c in range(4)]

    scores = cls_pred.reshape(_ROWS, 128)
    ancs_flat = ancs.reshape(_N, 4)
    anc_planes = [ancs_flat[:, c].reshape(_ROWS, 128) for c in range(4)]
    reg2 = [p.reshape(_ROWS, 128) for p in reg_planes]

    plane = lambda: jax.ShapeDtypeStruct((_ROWS, 128), jnp.float32)
    outs = pl.pallas_call(
        _nms_body,
        in_specs=[pl.BlockSpec(memory_space=pltpu.VMEM)] * 9,
        out_specs=[pl.BlockSpec(memory_space=pltpu.SMEM)] * 4,
        out_shape=[jax.ShapeDtypeStruct((_POST,), jnp.float32)] * 4,
        scratch_shapes=[pltpu.VMEM((_ROWS, 128), jnp.float32)] * 6,
    )(scores, *anc_planes, *reg2)

    proposals = jnp.stack(outs, axis=-1)
    cls_out = cls_pred.reshape(1, 64, 64, _NUM_ANCS)
    reg_out = jnp.stack(reg_planes, axis=-1).reshape(1, 64, 64, _NUM_ANCS, 4)
    return cls_out, reg_out, proposals


# speedup vs baseline: 14.2460x; 1.0631x over previous
import jax
import jax.numpy as jnp
from jax import lax
from jax.experimental import pallas as pl
from jax.experimental.pallas import tpu as pltpu

_NUM_ANCS = 9
_PRE = 6000
_POST = 300
_IOU_T = 0.7
_HW = 4096
_N = _HW * _NUM_ANCS
_ROWS = _N // 128
_TM = 512


def _mm_body(x_ref, w1_ref, b1_ref, wh_ref, bh_ref, y_ref):
    h = jnp.dot(x_ref[...], w1_ref[...], preferred_element_type=jnp.float32)
    h = jnp.maximum(h + b1_ref[...], 0.0)
    y = jnp.dot(h, wh_ref[...], preferred_element_type=jnp.float32) + bh_ref[...]
    y_ref[:, 0:128] = jax.nn.sigmoid(y[:, 0:128])
    y_ref[:, 128:640] = y[:, 128:640]


def _nms_body(s_ref, a0_ref, a1_ref, a2_ref, a3_ref, r0_ref, r1_ref, r2_ref,
              r3_ref, ox1_ref, oy1_ref, ox2_ref, oy2_ref,
              ms_ref, x1_ref, y1_ref, x2_ref, y2_ref, ar_ref):
    a2 = a2_ref[...]
    a3 = a3_ref[...]
    cx = a0_ref[...] + r0_ref[...] * a2
    cy = a1_ref[...] + r1_ref[...] * a3
    w = a2 * jnp.exp(r2_ref[...])
    h = a3 * jnp.exp(r3_ref[...])
    x1 = jnp.clip(cx - w * 0.5, 0.0, 1.0)
    y1 = jnp.clip(cy - h * 0.5, 0.0, 1.0)
    x2 = jnp.clip(cx + w * 0.5, 0.0, 1.0)
    y2 = jnp.clip(cy + h * 0.5, 0.0, 1.0)
    x1_ref[...] = x1
    y1_ref[...] = y1
    x2_ref[...] = x2
    y2_ref[...] = y2
    ar_ref[...] = (x2 - x1) * (y2 - y1)

    s = s_ref[...]
    bits = lax.bitcast_convert_type(s, jnp.int32)
    idx = (lax.broadcasted_iota(jnp.int32, (_ROWS, 128), 0) * 128
           + lax.broadcasted_iota(jnp.int32, (_ROWS, 128), 1))
    tau = jnp.int32(0)
    for b in range(30, -1, -1):
        cand = tau | jnp.int32(1 << b)
        cnt = jnp.sum((bits >= cand).astype(jnp.int32))
        tau = jnp.where(cnt >= _PRE, cand, tau)
    cnt_gt = jnp.sum((bits > tau).astype(jnp.int32))
    tie = bits == tau
    need = _PRE - cnt_gt
    theta = jnp.int32(0)
    for b in range(15, -1, -1):
        cand = theta | jnp.int32(1 << b)
        cnt = jnp.sum((tie & (idx >= cand)).astype(jnp.int32))
        theta = jnp.where(cnt >= need, cand, theta)
    active = (bits > tau) | (tie & (idx >= theta))
    ms_ref[...] = jnp.where(active, s, -1.0)

    def zero_body(k, c):
        ox1_ref[k] = 0.0
        oy1_ref[k] = 0.0
        ox2_ref[k] = 0.0
        oy2_ref[k] = 0.0
        return c

    lax.fori_loop(0, _POST, zero_body, 0)

    def nms_step(k, c):
        ms = ms_ref[...]
        m = jnp.max(ms)

        @pl.when(m >= 0.0)
        def _():
            bi = jnp.max(jnp.where(ms == m, idx, -1))
            br_ = bi // 128
            bc_ = bi % 128
            lane_eq = lax.broadcasted_iota(jnp.int32, (1, 128), 1) == bc_

            def _pick(ref):
                return jnp.sum(jnp.where(lane_eq, ref[pl.ds(br_, 1), :], 0.0))

            bx1 = _pick(x1_ref)
            by1 = _pick(y1_ref)
            bx2 = _pick(x2_ref)
            by2 = _pick(y2_ref)
            ix1 = jnp.maximum(x1_ref[...], bx1)
            iy1 = jnp.maximum(y1_ref[...], by1)
            ix2 = jnp.minimum(x2_ref[...], bx2)
            iy2 = jnp.minimum(y2_ref[...], by2)
            inter = (jnp.maximum(ix2 - ix1, 0.0) * jnp.maximum(iy2 - iy1, 0.0))
            barea = (bx2 - bx1) * (by2 - by1)
            union = jnp.maximum(barea + ar_ref[...] - inter, 1e-8)
            supp = inter > _IOU_T * union
            ms_ref[...] = jnp.where(supp, -1.0, ms)
            ox1_ref[k] = bx1
            oy1_ref[k] = by1
            ox2_ref[k] = bx2
            oy2_ref[k] = by2

        return c

    lax.fori_loop(0, _POST, nms_step, 0)


def kernel(feats, ancs, ancs_valid, W1, b1, Wc, bc, Wr, br):
    del ancs_valid
    x = feats.reshape(_HW, 1024)

    wh = jnp.zeros((512, 640), jnp.float32)
    bh = jnp.zeros((640,), jnp.float32)
    wh = wh.at[:, 0:_NUM_ANCS].set(Wc)
    bh = bh.at[0:_NUM_ANCS].set(bc)
    for c in range(4):
        wh = wh.at[:, 128 * (c + 1):128 * (c + 1) + _NUM_ANCS].set(Wr[:, c::4])
        bh = bh.at[128 * (c + 1):128 * (c + 1) + _NUM_ANCS].set(br[c::4])

    y = pl.pallas_call(
        _mm_body,
        grid=(_HW // _TM,),
        in_specs=[
            pl.BlockSpec((_TM, 1024), lambda i: (i, 0)),
            pl.BlockSpec((1024, 512), lambda i: (0, 0)),
            pl.BlockSpec((1, 512), lambda i: (0, 0)),
            pl.BlockSpec((512, 640), lambda i: (0, 0)),
            pl.BlockSpec((1, 640), lambda i: (0, 0)),
        ],
        out_specs=pl.BlockSpec((_TM, 640), lambda i: (i, 0)),
        out_shape=jax.ShapeDtypeStruct((_HW, 640), jnp.float32),
    )(x, W1, b1.reshape(1, 512), wh, bh.reshape(1, 640))

    cls_pred = y[:, 0:_NUM_ANCS]
    reg_planes = [y[:, 128 * (c + 1):128 * (c + 1) + _NUM_ANCS] for c in range(4)]

    scores = cls_pred.reshape(_ROWS, 128)
    ancs_flat = ancs.reshape(_N, 4)
    anc_planes = [ancs_flat[:, c].reshape(_ROWS, 128) for c in range(4)]
    reg2 = [p.reshape(_ROWS, 128) for p in reg_planes]

    outs = pl.pallas_call(
        _nms_body,
        in_specs=[pl.BlockSpec(memory_space=pltpu.VMEM)] * 9,
        out_specs=[pl.BlockSpec(memory_space=pltpu.SMEM)] * 4,
        out_shape=[jax.ShapeDtypeStruct((_POST,), jnp.float32)] * 4,
        scratch_shapes=[pltpu.VMEM((_ROWS, 128), jnp.float32)] * 6,
    )(scores, *anc_planes, *reg2)

    proposals = jnp.stack(outs, axis=-1)
    cls_out = cls_pred.reshape(1, 64, 64, _NUM_ANCS)
    reg_out = jnp.stack(reg_planes, axis=-1).reshape(1, 64, 64, _NUM_ANCS, 4)
    return cls_out, reg_out, proposals
